# baseline (speedup 1.0000x reference)
import jax
import jax.numpy as jnp
from jax import lax
from jax.experimental import pallas as pl
from jax.experimental.pallas import tpu as pltpu

TOP_K = 5
N = 1000000
NPAD = 1003520
ROWS = NPAD // 128
BLK = 560
GRID = ROWS // BLK


def _qnormalize(q):
    n = jnp.linalg.norm(q, axis=-1, keepdims=True)
    return q / jnp.maximum(n, 1e-12)


def _qconj(q):
    return jnp.concatenate([q[..., :1], -q[..., 1:]], axis=-1)


def _qmul(a, b):
    aw, ax, ay, az = a[..., 0], a[..., 1], a[..., 2], a[..., 3]
    bw, bx, by, bz = b[..., 0], b[..., 1], b[..., 2], b[..., 3]
    w = aw * bw - ax * bx - ay * by - az * bz
    x = aw * bx + ax * bw + ay * bz - az * by
    y = aw * by - ax * bz + ay * bw + az * bx
    z = aw * bz + ax * by - ay * bx + az * bw
    return jnp.stack([w, x, y, z], axis=-1)


def _score(params_ref, vw, vx, vy, vz):
    s2 = (vw * vw + vy * vy) + (vx * vx + vz * vz)
    n = jnp.maximum(jnp.sqrt(s2), jnp.float32(1e-12))
    qw = vw / n
    qx = vx / n
    qy = vy / n
    qz = vz / n
    mx = -qx
    my = -qy
    mz = -qz
    aw = params_ref[0]
    ax = params_ref[1]
    ay = params_ref[2]
    az = params_ref[3]
    bw = params_ref[4]
    bx = params_ref[5]
    by = params_ref[6]
    bz = params_ref[7]
    cw = params_ref[8]
    cx = params_ref[9]
    cy = params_ref[10]
    cz = params_ref[11]
    t1w = ((aw * qw - ax * mx) - ay * my) - az * mz
    t1x = ((aw * mx + ax * qw) + ay * mz) - az * my
    t1y = ((aw * my - ax * mz) + ay * qw) + az * mx
    t1z = ((aw * mz + ax * my) - ay * mx) + az * qw
    t2w = ((qw * bw - qx * bx) - qy * by) - qz * bz
    t2x = ((qw * bx + qx * bw) + qy * bz) - qz * by
    t2y = ((qw * by - qx * bz) + qy * bw) + qz * bx
    t2z = ((qw * bz + qx * by) - qy * bx) + qz * bw
    s1w = ((cw * t1w - cx * t1x) - cy * t1y) - cz * t1z
    s1x = ((cw * t1x + cx * t1w) + cy * t1z) - cz * t1y
    s1y = ((cw * t1y - cx * t1z) + cy * t1w) + cz * t1x
    s1z = ((cw * t1z + cx * t1y) - cy * t1x) + cz * t1w
    return ((s1w * t2w - s1x * t2x) - s1y * t2y) - s1z * t2z


def _merge_topk(sim, idxvec, keys_ref, idxs_ref, svals_ref):
    INT_MIN = jnp.int32(-2**31)
    INT_MAX = jnp.int32(2**31 - 1)
    b = lax.bitcast_convert_type(sim, jnp.int32)
    key = jnp.where(b < 0, (~b) ^ INT_MIN, b)
    kc = jnp.where(idxvec < N, key, INT_MIN)
    for _c in range(TOP_K):
        m = jnp.max(kc)
        sel = kc == m
        im = jnp.min(jnp.where(sel, idxvec, INT_MAX))
        sv = jnp.max(jnp.where(sel, sim, jnp.float32(-jnp.inf)))
        ks = [keys_ref[j] for j in range(TOP_K)]
        is_ = [idxs_ref[j] for j in range(TOP_K)]
        ss = [svals_ref[j] for j in range(TOP_K)]
        pos = jnp.int32(0)
        for j in range(TOP_K):
            pos = pos + jnp.where(ks[j] >= m, jnp.int32(1), jnp.int32(0))
        for j in range(TOP_K - 1, -1, -1):
            if j == 0:
                nk = jnp.where(pos > 0, ks[0], m)
                ni = jnp.where(pos > 0, is_[0], im)
                ns = jnp.where(pos > 0, ss[0], sv)
            else:
                nk = jnp.where(pos > j, ks[j], jnp.where(pos == j, m, ks[j - 1]))
                ni = jnp.where(pos > j, is_[j], jnp.where(pos == j, im, is_[j - 1]))
                ns = jnp.where(pos > j, ss[j], jnp.where(pos == j, sv, ss[j - 1]))
            keys_ref[j] = nk
            idxs_ref[j] = ni
            svals_ref[j] = ns
        kc = jnp.where(sel & (idxvec == im), INT_MIN, kc)


def _body(params_ref, w_ref, x_ref, y_ref, z_ref,
          scores_out, idx_out, keys_ref, idxs_ref, svals_ref):
    i = pl.program_id(0)

    @pl.when(i == 0)
    def _init():
        for j in range(TOP_K):
            keys_ref[j] = jnp.int32(-2**31)
            idxs_ref[j] = jnp.int32(0)
            svals_ref[j] = jnp.float32(-jnp.inf)

    sim = _score(params_ref, w_ref[...], x_ref[...], y_ref[...], z_ref[...])
    bm = jnp.max(sim)

    @pl.when(bm > svals_ref[TOP_K - 1])
    def _merge():
        si = lax.broadcasted_iota(jnp.int32, (BLK, 128), 0)
        li = lax.broadcasted_iota(jnp.int32, (BLK, 128), 1)
        idxvec = i * (BLK * 128) + si * 128 + li
        _merge_topk(sim, idxvec, keys_ref, idxs_ref, svals_ref)

    @pl.when(i == GRID - 1)
    def _emit():
        for j in range(TOP_K):
            scores_out[j] = svals_ref[j]
            idx_out[j] = idxs_ref[j]


def kernel(x, context, vault_knots):
    q_now = _qnormalize(x)
    q_ctx = _qnormalize(context)
    c12 = _qmul(q_now[None, :], _qconj(q_ctx)[None, :])[0]
    params = jnp.concatenate([q_ctx, _qconj(q_now), c12])

    comps = []
    for c in range(4):
        col = lax.slice(vault_knots, (0, c), (N, c + 1))
        col = jnp.pad(col, ((0, NPAD - N), (0, 0)))
        comps.append(col.reshape(ROWS, 128))

    blk = pl.BlockSpec((BLK, 128), lambda i: (i, 0))
    scores, idx = pl.pallas_call(
        _body,
        grid=(GRID,),
        in_specs=[pl.BlockSpec(memory_space=pltpu.SMEM)] + [blk] * 4,
        out_specs=[pl.BlockSpec(memory_space=pltpu.SMEM),
                   pl.BlockSpec(memory_space=pltpu.SMEM)],
        out_shape=[jax.ShapeDtypeStruct((TOP_K,), jnp.float32),
                   jax.ShapeDtypeStruct((TOP_K,), jnp.int32)],
        scratch_shapes=[pltpu.SMEM((TOP_K,), jnp.int32),
                        pltpu.SMEM((TOP_K,), jnp.int32),
                        pltpu.SMEM((TOP_K,), jnp.float32)],
    )(params, *comps)

    recalled = vault_knots.at[idx].get(mode="promise_in_bounds")
    return (recalled, scores)


# speedup vs baseline: 1.2466x; 1.2466x over previous
import jax
import jax.numpy as jnp
from jax import lax
from jax.experimental import pallas as pl
from jax.experimental.pallas import tpu as pltpu

TOP_K = 5
N = 1000000
NMAIN = 999424
ROWS = NMAIN // 128
BLK = 488
GRID = ROWS // BLK
NTAIL = N - NMAIN


def _qnormalize(q):
    n = jnp.linalg.norm(q, axis=-1, keepdims=True)
    return q / jnp.maximum(n, 1e-12)


def _qconj(q):
    return jnp.concatenate([q[..., :1], -q[..., 1:]], axis=-1)


def _qmul(a, b):
    aw, ax, ay, az = a[..., 0], a[..., 1], a[..., 2], a[..., 3]
    bw, bx, by, bz = b[..., 0], b[..., 1], b[..., 2], b[..., 3]
    w = aw * bw - ax * bx - ay * by - az * bz
    x = aw * bx + ax * bw + ay * bz - az * by
    y = aw * by - ax * bz + ay * bw + az * bx
    z = aw * bz + ax * by - ay * bx + az * bw
    return jnp.stack([w, x, y, z], axis=-1)


def _score(params_ref, vw, vx, vy, vz):
    s2 = (vw * vw + vy * vy) + (vx * vx + vz * vz)
    n = jnp.maximum(jnp.sqrt(s2), jnp.float32(1e-12))
    qw = vw / n
    qx = vx / n
    qy = vy / n
    qz = vz / n
    mx = -qx
    my = -qy
    mz = -qz
    aw = params_ref[0]
    ax = params_ref[1]
    ay = params_ref[2]
    az = params_ref[3]
    bw = params_ref[4]
    bx = params_ref[5]
    by = params_ref[6]
    bz = params_ref[7]
    cw = params_ref[8]
    cx = params_ref[9]
    cy = params_ref[10]
    cz = params_ref[11]
    t1w = ((aw * qw - ax * mx) - ay * my) - az * mz
    t1x = ((aw * mx + ax * qw) + ay * mz) - az * my
    t1y = ((aw * my - ax * mz) + ay * qw) + az * mx
    t1z = ((aw * mz + ax * my) - ay * mx) + az * qw
    t2w = ((qw * bw - qx * bx) - qy * by) - qz * bz
    t2x = ((qw * bx + qx * bw) + qy * bz) - qz * by
    t2y = ((qw * by - qx * bz) + qy * bw) + qz * bx
    t2z = ((qw * bz + qx * by) - qy * bx) + qz * bw
    s1w = ((cw * t1w - cx * t1x) - cy * t1y) - cz * t1z
    s1x = ((cw * t1x + cx * t1w) + cy * t1z) - cz * t1y
    s1y = ((cw * t1y - cx * t1z) + cy * t1w) + cz * t1x
    s1z = ((cw * t1z + cx * t1y) - cy * t1x) + cz * t1w
    return ((s1w * t2w - s1x * t2x) - s1y * t2y) - s1z * t2z


def _merge_topk(sim, idxvec, keys_ref, idxs_ref, svals_ref):
    INT_MIN = jnp.int32(-2**31)
    INT_MAX = jnp.int32(2**31 - 1)
    b = lax.bitcast_convert_type(sim, jnp.int32)
    key = jnp.where(b < 0, (~b) ^ INT_MIN, b)
    kc = jnp.where(idxvec < N, key, INT_MIN)
    for _c in range(TOP_K):
        m = jnp.max(kc)
        sel = kc == m
        im = jnp.min(jnp.where(sel, idxvec, INT_MAX))
        sv = jnp.max(jnp.where(sel, sim, jnp.float32(-jnp.inf)))
        ks = [keys_ref[j] for j in range(TOP_K)]
        is_ = [idxs_ref[j] for j in range(TOP_K)]
        ss = [svals_ref[j] for j in range(TOP_K)]
        pos = jnp.int32(0)
        for j in range(TOP_K):
            pos = pos + jnp.where(ks[j] >= m, jnp.int32(1), jnp.int32(0))
        for j in range(TOP_K - 1, -1, -1):
            if j == 0:
                nk = jnp.where(pos > 0, ks[0], m)
                ni = jnp.where(pos > 0, is_[0], im)
                ns = jnp.where(pos > 0, ss[0], sv)
            else:
                nk = jnp.where(pos > j, ks[j], jnp.where(pos == j, m, ks[j - 1]))
                ni = jnp.where(pos > j, is_[j], jnp.where(pos == j, im, is_[j - 1]))
                ns = jnp.where(pos > j, ss[j], jnp.where(pos == j, sv, ss[j - 1]))
            keys_ref[j] = nk
            idxs_ref[j] = ni
            svals_ref[j] = ns
        kc = jnp.where(sel & (idxvec == im), INT_MIN, kc)


def _body(params_ref, w_ref, x_ref, y_ref, z_ref,
          tw_ref, tx_ref, ty_ref, tz_ref,
          scores_out, idx_out, keys_ref, idxs_ref, svals_ref):
    i = pl.program_id(0)

    @pl.when(i == 0)
    def _init():
        for j in range(TOP_K):
            keys_ref[j] = jnp.int32(-2**31)
            idxs_ref[j] = jnp.int32(0)
            svals_ref[j] = jnp.float32(-jnp.inf)

    sim = _score(params_ref, w_ref[...], x_ref[...], y_ref[...], z_ref[...])
    bm = jnp.max(sim)

    @pl.when(bm > svals_ref[TOP_K - 1])
    def _merge():
        si = lax.broadcasted_iota(jnp.int32, (BLK, 128), 0)
        li = lax.broadcasted_iota(jnp.int32, (BLK, 128), 1)
        idxvec = i * (BLK * 128) + si * 128 + li
        _merge_topk(sim, idxvec, keys_ref, idxs_ref, svals_ref)

    @pl.when(i == GRID - 1)
    def _tail_and_emit():
        tsim = _score(params_ref, tw_ref[...], tx_ref[...], ty_ref[...], tz_ref[...])
        tbm = jnp.max(tsim)

        @pl.when(tbm > svals_ref[TOP_K - 1])
        def _tmerge():
            si = lax.broadcasted_iota(jnp.int32, (8, 128), 0)
            li = lax.broadcasted_iota(jnp.int32, (8, 128), 1)
            idxvec = NMAIN + si * 128 + li
            _merge_topk(tsim, idxvec, keys_ref, idxs_ref, svals_ref)

        for j in range(TOP_K):
            scores_out[j] = svals_ref[j]
            idx_out[j] = idxs_ref[j]


def kernel(x, context, vault_knots):
    q_now = _qnormalize(x)
    q_ctx = _qnormalize(context)
    c12 = _qmul(q_now[None, :], _qconj(q_ctx)[None, :])[0]
    params = jnp.concatenate([q_ctx, _qconj(q_now), c12])

    tail2d = lax.slice(vault_knots, (NMAIN, 0), (N, 4))
    comps = []
    tails = []
    for c in range(4):
        comps.append(lax.slice(vault_knots, (0, c), (NMAIN, c + 1)).reshape(ROWS, 128))
        tails.append(jnp.pad(tail2d[:, c], (0, 1024 - NTAIL)).reshape(8, 128))

    blk = pl.BlockSpec((BLK, 128), lambda i: (i, 0))
    tblk = pl.BlockSpec((8, 128), lambda i: (0, 0))
    scores, idx = pl.pallas_call(
        _body,
        grid=(GRID,),
        in_specs=[pl.BlockSpec(memory_space=pltpu.SMEM)] + [blk] * 4 + [tblk] * 4,
        out_specs=[pl.BlockSpec(memory_space=pltpu.SMEM),
                   pl.BlockSpec(memory_space=pltpu.SMEM)],
        out_shape=[jax.ShapeDtypeStruct((TOP_K,), jnp.float32),
                   jax.ShapeDtypeStruct((TOP_K,), jnp.int32)],
        scratch_shapes=[pltpu.SMEM((TOP_K,), jnp.int32),
                        pltpu.SMEM((TOP_K,), jnp.int32),
                        pltpu.SMEM((TOP_K,), jnp.float32)],
    )(params, *comps, *tails)

    recalled = vault_knots.at[idx].get(mode="promise_in_bounds")
    return (recalled, scores)
